# Initial kernel scaffold; baseline (speedup 1.0000x reference)
import jax
import jax.numpy as jnp
from jax.experimental import pallas as pl


def kernel(x, enc_w1, enc_b1, enc_w2, enc_b2, enc_w3, enc_b3, dec_w1, dec_b1, dec_wt1, dec_bt1, dec_wt2, dec_bt2, embedding, cluster_size):
    raise NotImplementedError("write your pallas kernel here")


# speedup vs baseline: 1.6583x; 1.6583x over previous
import functools

import jax
import jax.numpy as jnp
from jax import lax
from jax.experimental import pallas as pl
from jax.experimental.pallas import tpu as pltpu
from jax.experimental.pallas import tpu_sc as plsc

B, T = 8, 32768
K, D = 2048, 128
TQ = 2048
T2 = 8192
VQ_CHUNK = 512


_INV_SQRT2 = 0.7071067811865476

_ERF_P = [7.85386146e-05, -0.000801019371, 0.00518832775, -0.0268538129,
          0.112835854, -0.37612626, 1.12837911]
_ERFC_Q2 = [0.0232682, -0.138703942, 0.368742466, -0.582473278, 0.621000469,
            -0.494451523, 0.340488, -0.274112701, 0.563825965]
_ERFC_QL = [-10.477664, 12.9772, -7.49551868, 2.92101908, -1.01526523,
            0.42184633, -0.282076746, 0.564189494]


def _gelu(v):
    f32 = jnp.float32
    half_v = v * f32(0.5)
    x = (-v) * f32(0.707106769)
    ax = jnp.abs(x)
    x2 = x * x
    p = jnp.full_like(v, _ERF_P[0])
    for c in _ERF_P[1:]:
        p = p * x2 + f32(c)
    erfc_small = f32(1.0) - x * p
    nx2 = -x2
    ex = jnp.exp(nx2)
    exr = ex * (f32(1.0) / ax)
    z = f32(1.0) / x2
    q2 = jnp.full_like(v, _ERFC_Q2[0])
    for c in _ERFC_Q2[1:]:
        q2 = q2 * z + f32(c)
    ql = z * f32(_ERFC_QL[0]) + f32(_ERFC_QL[1])
    for c in _ERFC_QL[2:]:
        ql = ql * z + f32(c)
    qq = jnp.where(ax < f32(2.0), q2, ql)
    res = exr * qq
    res = jnp.where(nx2 < f32(-88.7228394), f32(0.0), res)
    res = jnp.where(x < f32(0.0), f32(2.0) - res, res)
    erfc_val = jnp.where(ax < f32(1.0), erfc_small, res)
    return half_v * erfc_val


def _encoder_body(xfp_ref, w1t_ref, b1_ref, w2c_ref, b2_ref, w3c_ref, b3_ref,
                  embT_ref, z_ref, idx_ref):
    xfp = xfp_ref[0]

    phases = []
    for j in range(4):
        cols = []
        for k in range(8):
            r, c = divmod(4 * j + k - 10, 16)
            cols.append(xfp[r + 1:r + 2050, c:c + 1])
        x1 = jnp.concatenate(cols, axis=1)
        acc = jnp.dot(x1.astype(jnp.bfloat16), w1t_ref[...],
                      preferred_element_type=jnp.float32)
        acc = _gelu(acc + b1_ref[...])
        row = lax.broadcasted_iota(jnp.int32, (2049, 128), 0)
        bad = (row == 0) if j < 2 else (row == 2048)
        phases.append(jnp.where(bad, 0.0, acc))

    x2 = jnp.concatenate(
        [phases[j][q:q + 2048, :] for q in range(2) for j in range(4)],
        axis=1)
    acc2 = None
    for ch in range(4):
        t = jnp.dot(x2[:, ch * 256:(ch + 1) * 256].astype(jnp.bfloat16),
                    w2c_ref[ch * 256:(ch + 1) * 256, :],
                    preferred_element_type=jnp.float32)
        acc2 = t if acc2 is None else acc2 + t
    g2 = _gelu(acc2 + b2_ref[...])

    zpad = jnp.concatenate(
        [jnp.zeros((1, 128), jnp.float32), g2, jnp.zeros((1, 128), jnp.float32)],
        axis=0)
    x3 = jnp.concatenate([zpad[m:m + 2048, :] for m in range(3)], axis=1)
    z = jnp.dot(x3.astype(jnp.bfloat16), w3c_ref[...],
                preferred_element_type=jnp.float32) + b3_ref[...]
    z_ref[0] = z

    embT = embT_ref[...]
    enorm = jnp.sqrt(jnp.sum(embT * embT, axis=0, keepdims=True))
    enT = embT / jnp.maximum(enorm, 1e-12)
    for cstart in range(0, TQ, VQ_CHUNK):
        zc = z[cstart:cstart + VQ_CHUNK, :]
        znorm = jnp.sqrt(jnp.sum(zc * zc, axis=1, keepdims=True))
        zn = zc / jnp.maximum(znorm, 1e-12)
        sim = jnp.dot(zn.astype(jnp.bfloat16), enT.astype(jnp.bfloat16),
                      preferred_element_type=jnp.float32)
        maxv = jnp.max(sim, axis=1, keepdims=True)
        kio = lax.broadcasted_iota(jnp.int32, (VQ_CHUNK, K), 1)
        idxc = jnp.min(jnp.where(sim == maxv, kio, K), axis=1, keepdims=True)
        idx_ref[0, pl.ds(cstart, VQ_CHUNK), :] = idxc


def _encoder_call(xfp, w1t, b1, w2c, b2, w3c, b3, embT):
    full = lambda a: pl.BlockSpec(a.shape, lambda b: (0,) * a.ndim)
    return pl.pallas_call(
        _encoder_body,
        grid=(B,),
        in_specs=[
            pl.BlockSpec((1, 2050, 16), lambda b: (b, 0, 0)),
            full(w1t), full(b1), full(w2c), full(b2), full(w3c), full(b3),
            full(embT),
        ],
        out_specs=[
            pl.BlockSpec((1, TQ, D), lambda b: (b, 0, 0)),
            pl.BlockSpec((1, TQ, 1), lambda b: (b, 0, 0)),
        ],
        out_shape=[
            jax.ShapeDtypeStruct((B, TQ, D), jnp.float32),
            jax.ShapeDtypeStruct((B, TQ, 1), jnp.int32),
        ],
    )(xfp, w1t, b1, w2c, b2, w3c, b3, embT)


_SC_NC, _SC_NS = 2, 16
_NW = _SC_NC * _SC_NS
_ROWS_PER_W = (B * TQ) // _NW


def _gather_body(table_hbm, idx_hbm, out_hbm, idx_v, rows_v, sem):
    wid = lax.axis_index("s") * _SC_NC + lax.axis_index("c")
    base = wid * _ROWS_PER_W
    pltpu.sync_copy(idx_hbm.at[pl.ds(base, _ROWS_PER_W)], idx_v)
    pltpu.async_copy(table_hbm.at[idx_v], rows_v, sem).wait()
    pltpu.sync_copy(rows_v, out_hbm.at[pl.ds(base, _ROWS_PER_W)])


def _gather_call(table, idx_flat):
    k = functools.partial(
        pl.kernel,
        out_type=jax.ShapeDtypeStruct((B * TQ, D), jnp.float32),
        mesh=plsc.VectorSubcoreMesh(core_axis_name="c", subcore_axis_name="s"),
        scratch_types=[
            pltpu.VMEM((_ROWS_PER_W,), jnp.int32),
            pltpu.VMEM((_ROWS_PER_W, D), jnp.float32),
            pltpu.SemaphoreType.DMA,
        ],
    )(_gather_body)
    return k(table, idx_flat)


def _decoder_body(zq_ref, z_ref, xf_ref, wd1c_ref, bd1_ref, w01_ref, w23_ref,
                  bt1_ref, wt2a_ref, wt2b_ref, bt2_ref, cs_ref,
                  xr_ref, rec_ref, com_ref, ppx_ref):
    b = pl.program_id(0)
    zq = zq_ref[0]

    zqp = jnp.concatenate(
        [jnp.zeros((1, 128), jnp.float32), zq, jnp.zeros((1, 128), jnp.float32)],
        axis=0)
    xd = jnp.concatenate([zqp[m:m + 2048, :] for m in range(3)], axis=1)
    h = _gelu(jnp.dot(xd.astype(jnp.bfloat16), wd1c_ref[...],
                      preferred_element_type=jnp.float32) + bd1_ref[...])

    hp = jnp.concatenate(
        [jnp.zeros((1, 128), jnp.float32), h, jnp.zeros((1, 128), jnp.float32)],
        axis=0)
    x01 = jnp.concatenate([h, hp[0:2048, :]], axis=1)
    h01 = _gelu(jnp.dot(x01.astype(jnp.bfloat16), w01_ref[...],
                        preferred_element_type=jnp.float32) + bt1_ref[...])
    x23 = jnp.concatenate([h, hp[2:2050, :]], axis=1)
    h23 = _gelu(jnp.dot(x23.astype(jnp.bfloat16), w23_ref[...],
                        preferred_element_type=jnp.float32) + bt1_ref[...])
    h2 = [h01[:, 0:128], h01[:, 128:256], h23[:, 0:128], h23[:, 128:256]]

    wt2a = wt2a_ref[...]
    wt2b = wt2b_ref[...]
    zrow = jnp.zeros((1, 128), jnp.float32)
    blocks = []
    for j in range(4):
        prev = (h2[j - 1] if j > 0
                else jnp.concatenate([zrow, h2[3][0:2047, :]], axis=0))
        nxt = (h2[j + 1] if j < 3
               else jnp.concatenate([h2[0][1:2048, :], zrow], axis=0))
        blk = (jnp.dot(h2[j].astype(jnp.bfloat16), wt2a,
                       preferred_element_type=jnp.float32)
               + jnp.concatenate(
                   [jnp.dot(prev.astype(jnp.bfloat16), wt2b[:, 0:2],
                            preferred_element_type=jnp.float32),
                    jnp.dot(nxt.astype(jnp.bfloat16), wt2b[:, 2:4],
                            preferred_element_type=jnp.float32)],
                   axis=1))
        blocks.append(blk)
    xr = jnp.tanh(jnp.concatenate(blocks, axis=1) + bt2_ref[0, 0])
    xr_ref[0] = xr

    @pl.when(b == 0)
    def _init():
        rec_ref[...] = jnp.zeros((1, 1), jnp.float32)
        com_ref[...] = jnp.zeros((1, 1), jnp.float32)
        cs = cs_ref[...]
        n = cs / (jnp.sum(cs) + 1e-6)
        ppx_ref[...] = jnp.exp(-jnp.sum(n * jnp.log(n + 1e-6),
                                        keepdims=True))

    dr = xr - xf_ref[0]
    rec_ref[...] += jnp.sum(dr * dr, keepdims=True)
    dc = z_ref[0] - zq
    com_ref[...] += jnp.sum(dc * dc, keepdims=True)


def _decoder_call(zq, z, xf, wd1c, bd1, w01, w23, bt1, wt2a, wt2b, bt2, cs):
    full = lambda a: pl.BlockSpec(a.shape, lambda b: (0,) * a.ndim)
    return pl.pallas_call(
        _decoder_body,
        grid=(B,),
        in_specs=[
            pl.BlockSpec((1, TQ, D), lambda b: (b, 0, 0)),
            pl.BlockSpec((1, TQ, D), lambda b: (b, 0, 0)),
            pl.BlockSpec((1, 2048, 16), lambda b: (b, 0, 0)),
            full(wd1c), full(bd1), full(w01), full(w23), full(bt1),
            full(wt2a), full(wt2b), full(bt2), full(cs),
        ],
        out_specs=[
            pl.BlockSpec((1, 2048, 16), lambda b: (b, 0, 0)),
            pl.BlockSpec((1, 1), lambda b: (0, 0)),
            pl.BlockSpec((1, 1), lambda b: (0, 0)),
            pl.BlockSpec((1, 1), lambda b: (0, 0)),
        ],
        out_shape=[
            jax.ShapeDtypeStruct((B, 2048, 16), jnp.float32),
            jax.ShapeDtypeStruct((1, 1), jnp.float32),
            jax.ShapeDtypeStruct((1, 1), jnp.float32),
            jax.ShapeDtypeStruct((1, 1), jnp.float32),
        ],
    )(zq, z, xf, wd1c, bd1, w01, w23, bt1, wt2a, wt2b, bt2, cs)


def kernel(x, enc_w1, enc_b1, enc_w2, enc_b2, enc_w3, enc_b3,
           dec_w1, dec_b1, dec_wt1, dec_bt1, dec_wt2, dec_bt2,
           embedding, cluster_size):
    f32 = jnp.float32

    bf = jnp.bfloat16
    xf = x.reshape(B, 2048, 16)
    xfp = jnp.pad(xf, ((0, 0), (1, 1), (0, 0)))
    w1t = enc_w1[:, 0, :].T.astype(bf)
    w2c = enc_w2.transpose(2, 1, 0).reshape(8 * 128, 128).astype(bf)
    w3c = enc_w3.transpose(2, 1, 0).reshape(3 * 128, 128).astype(bf)
    embT = embedding.T
    b1 = enc_b1.reshape(1, 128)
    b2 = enc_b2.reshape(1, 128)
    b3 = enc_b3.reshape(1, 128)

    wd1c = dec_w1.transpose(2, 1, 0).reshape(3 * 128, 128).astype(bf)
    bd1 = dec_b1.reshape(1, 128)
    w01 = jnp.concatenate(
        [jnp.concatenate([dec_wt1[:, :, 2], dec_wt1[:, :, 3]], axis=1),
         jnp.concatenate([dec_wt1[:, :, 6], dec_wt1[:, :, 7]], axis=1)], axis=0)
    w01 = w01.astype(bf)
    w23 = jnp.concatenate(
        [jnp.concatenate([dec_wt1[:, :, 4], dec_wt1[:, :, 5]], axis=1),
         jnp.concatenate([dec_wt1[:, :, 0], dec_wt1[:, :, 1]], axis=1)], axis=0)
    w23 = w23.astype(bf)
    bt1 = jnp.concatenate([dec_bt1, dec_bt1]).reshape(1, 256)
    wt2a = jnp.stack([dec_wt2[:, 0, 2], dec_wt2[:, 0, 3],
                      dec_wt2[:, 0, 4], dec_wt2[:, 0, 5]], axis=1).astype(bf)
    wt2b = jnp.stack([dec_wt2[:, 0, 6], dec_wt2[:, 0, 7],
                      dec_wt2[:, 0, 0], dec_wt2[:, 0, 1]], axis=1).astype(bf)
    bt2 = dec_bt2.reshape(1, 1)
    cs = cluster_size.reshape(1, K)

    z, idx3 = _encoder_call(xfp, w1t, b1, w2c, b2, w3c, b3, embT)
    idx_flat = idx3.reshape(B * TQ)

    zq_flat = _gather_call(embedding, idx_flat)
    zq = zq_flat.reshape(B, TQ, D)

    xr_f, rec_s, com_s, ppx_s = _decoder_call(
        zq, z, xf, wd1c, bd1, w01, w23, bt1, wt2a, wt2b, bt2, cs)

    xr = xr_f.reshape(B, 1, T)
    rec = (rec_s / f32(B * T)).reshape(())
    com = (com_s / f32(B * D * TQ)).reshape(())
    ppx = ppx_s.reshape(())
    idx = idx3.reshape(B, TQ)
    return (rec, com, ppx, xr, idx)


# speedup vs baseline: 2.2520x; 1.3580x over previous
import functools

import jax
import jax.numpy as jnp
from jax import lax
from jax.experimental import pallas as pl
from jax.experimental.pallas import tpu as pltpu
from jax.experimental.pallas import tpu_sc as plsc

B, T = 8, 32768
K, D = 2048, 128
TQ = 2048
T2 = 8192
VQ_CHUNK = 512


_INV_SQRT2 = 0.7071067811865476

_ERF_P = [7.85386146e-05, -0.000801019371, 0.00518832775, -0.0268538129,
          0.112835854, -0.37612626, 1.12837911]
_ERFC_Q2 = [0.0232682, -0.138703942, 0.368742466, -0.582473278, 0.621000469,
            -0.494451523, 0.340488, -0.274112701, 0.563825965]
_ERFC_QL = [-10.477664, 12.9772, -7.49551868, 2.92101908, -1.01526523,
            0.42184633, -0.282076746, 0.564189494]


def _gelu_fast(v):
    return 0.5 * v * (1.0 + lax.erf(v * _INV_SQRT2))


def _gelu(v):
    f32 = jnp.float32
    half_v = v * f32(0.5)
    x = (-v) * f32(0.707106769)
    ax = jnp.abs(x)
    x2 = x * x
    p = jnp.full_like(v, _ERF_P[0])
    for c in _ERF_P[1:]:
        p = p * x2 + f32(c)
    erfc_small = f32(1.0) - x * p
    nx2 = -x2
    ex = jnp.exp(nx2)
    exr = ex * (f32(1.0) / ax)
    z = f32(1.0) / x2
    q2 = jnp.full_like(v, _ERFC_Q2[0])
    for c in _ERFC_Q2[1:]:
        q2 = q2 * z + f32(c)
    ql = z * f32(_ERFC_QL[0]) + f32(_ERFC_QL[1])
    for c in _ERFC_QL[2:]:
        ql = ql * z + f32(c)
    qq = jnp.where(ax < f32(2.0), q2, ql)
    res = exr * qq
    res = jnp.where(nx2 < f32(-88.7228394), f32(0.0), res)
    res = jnp.where(x < f32(0.0), f32(2.0) - res, res)
    erfc_val = jnp.where(ax < f32(1.0), erfc_small, res)
    return half_v * erfc_val


def _encoder_body(xfp_ref, w1t_ref, b1_ref, w2c_ref, b2_ref, w3c_ref, b3_ref,
                  embT_ref, z_ref, idx_ref):
    xfp = xfp_ref[0]

    phases = []
    for j in range(4):
        cols = []
        for k in range(8):
            r, c = divmod(4 * j + k - 10, 16)
            cols.append(xfp[r + 1:r + 2050, c:c + 1])
        x1 = jnp.concatenate(cols, axis=1)
        acc = jnp.dot(x1.astype(jnp.bfloat16), w1t_ref[...],
                      preferred_element_type=jnp.float32)
        acc = _gelu(acc + b1_ref[...])
        row = lax.broadcasted_iota(jnp.int32, (2049, 128), 0)
        bad = (row == 0) if j < 2 else (row == 2048)
        phases.append(jnp.where(bad, 0.0, acc))

    x2 = jnp.concatenate(
        [phases[j][q:q + 2048, :] for q in range(2) for j in range(4)],
        axis=1)
    acc2 = None
    for ch in range(4):
        t = jnp.dot(x2[:, ch * 256:(ch + 1) * 256].astype(jnp.bfloat16),
                    w2c_ref[ch * 256:(ch + 1) * 256, :],
                    preferred_element_type=jnp.float32)
        acc2 = t if acc2 is None else acc2 + t
    g2 = _gelu(acc2 + b2_ref[...])

    zb = jnp.zeros((1, 128), jnp.float32)
    zpad = jnp.concatenate([zb, g2, zb], axis=0)
    x3 = jnp.concatenate([zpad[m:m + 2048, :] for m in range(3)], axis=1)
    z = jnp.dot(x3.astype(jnp.bfloat16), w3c_ref[...],
                preferred_element_type=jnp.float32) + b3_ref[...]
    z_ref[0] = z

    embT = embT_ref[...]
    enorm = jnp.sqrt(jnp.sum(embT * embT, axis=0, keepdims=True))
    enT = embT / jnp.maximum(enorm, 1e-12)
    for cstart in range(0, TQ, VQ_CHUNK):
        zc = z[cstart:cstart + VQ_CHUNK, :]
        znorm = jnp.sqrt(jnp.sum(zc * zc, axis=1, keepdims=True))
        zn = zc / jnp.maximum(znorm, 1e-12)
        sim = jnp.dot(zn.astype(jnp.bfloat16), enT.astype(jnp.bfloat16),
                      preferred_element_type=jnp.float32)
        maxv = jnp.max(sim, axis=1, keepdims=True)
        kio = lax.broadcasted_iota(jnp.int32, (VQ_CHUNK, K), 1)
        idxc = jnp.min(jnp.where(sim == maxv, kio, K), axis=1, keepdims=True)
        idx_ref[0, pl.ds(cstart, VQ_CHUNK), :] = idxc


def _encoder_call(xfp, w1t, b1, w2c, b2, w3c, b3, embT):
    full = lambda a: pl.BlockSpec(a.shape, lambda b: (0,) * a.ndim)
    return pl.pallas_call(
        _encoder_body,
        grid=(B,),
        in_specs=[
            pl.BlockSpec((1, 2050, 16), lambda b: (b, 0, 0)),
            full(w1t), full(b1), full(w2c), full(b2), full(w3c), full(b3),
            full(embT),
        ],
        out_specs=[
            pl.BlockSpec((1, TQ, D), lambda b: (b, 0, 0)),
            pl.BlockSpec((1, TQ, 1), lambda b: (b, 0, 0)),
        ],
        out_shape=[
            jax.ShapeDtypeStruct((B, TQ, D), jnp.float32),
            jax.ShapeDtypeStruct((B, TQ, 1), jnp.int32),
        ],
    )(xfp, w1t, b1, w2c, b2, w3c, b3, embT)


_SC_NC, _SC_NS = 2, 16
_NW = _SC_NC * _SC_NS
_ROWS_PER_W = (B * TQ) // _NW


def _gather_body(table_hbm, idx_hbm, out_hbm, idx_v, rows_v, sem):
    wid = lax.axis_index("s") * _SC_NC + lax.axis_index("c")
    base = wid * _ROWS_PER_W
    pltpu.sync_copy(idx_hbm.at[pl.ds(base, _ROWS_PER_W)], idx_v)
    pltpu.async_copy(table_hbm.at[idx_v], rows_v, sem).wait()
    pltpu.sync_copy(rows_v, out_hbm.at[pl.ds(base, _ROWS_PER_W)])


def _gather_call(table, idx_flat):
    k = functools.partial(
        pl.kernel,
        out_type=jax.ShapeDtypeStruct((B * TQ, D), jnp.float32),
        mesh=plsc.VectorSubcoreMesh(core_axis_name="c", subcore_axis_name="s"),
        scratch_types=[
            pltpu.VMEM((_ROWS_PER_W,), jnp.int32),
            pltpu.VMEM((_ROWS_PER_W, D), jnp.float32),
            pltpu.SemaphoreType.DMA,
        ],
    )(_gather_body)
    return k(table, idx_flat)


def _decoder_body(zq_ref, z_ref, xf_ref, wd1c_ref, bd1_ref, w01_ref, w23_ref,
                  bt1_ref, wt2a_ref, wt2b_ref, bt2_ref, cs_ref,
                  xr_ref, rec_ref, com_ref, ppx_ref):
    b = pl.program_id(0)
    zq = zq_ref[0]

    zqb = zq.astype(jnp.bfloat16)
    zbr = jnp.zeros((1, 128), jnp.bfloat16)
    zqp = jnp.concatenate([zbr, zqb, zbr], axis=0)
    xd = jnp.concatenate([zqp[m:m + 2048, :] for m in range(3)], axis=1)
    h = _gelu_fast(jnp.dot(xd, wd1c_ref[...],
                           preferred_element_type=jnp.float32) + bd1_ref[...])
    hb = h.astype(jnp.bfloat16)

    hp = jnp.concatenate([zbr, hb, zbr], axis=0)
    x01 = jnp.concatenate([hb, hp[0:2048, :]], axis=1)
    h01 = _gelu_fast(jnp.dot(x01, w01_ref[...],
                             preferred_element_type=jnp.float32) + bt1_ref[...])
    x23 = jnp.concatenate([hb, hp[2:2050, :]], axis=1)
    h23 = _gelu_fast(jnp.dot(x23, w23_ref[...],
                             preferred_element_type=jnp.float32) + bt1_ref[...])
    h2 = [h01[:, 0:128].astype(jnp.bfloat16), h01[:, 128:256].astype(jnp.bfloat16),
          h23[:, 0:128].astype(jnp.bfloat16), h23[:, 128:256].astype(jnp.bfloat16)]

    wt2a = wt2a_ref[...]
    wt2b = wt2b_ref[...]
    zrow = jnp.zeros((1, 128), jnp.bfloat16)
    blocks = []
    for j in range(4):
        prev = (h2[j - 1] if j > 0
                else jnp.concatenate([zrow, h2[3][0:2047, :]], axis=0))
        nxt = (h2[j + 1] if j < 3
               else jnp.concatenate([h2[0][1:2048, :], zrow], axis=0))
        blk = (jnp.dot(h2[j], wt2a, preferred_element_type=jnp.float32)
               + jnp.concatenate(
                   [jnp.dot(prev, wt2b[:, 0:2], preferred_element_type=jnp.float32),
                    jnp.dot(nxt, wt2b[:, 2:4], preferred_element_type=jnp.float32)],
                   axis=1))
        blocks.append(blk)
    xr = jnp.tanh(jnp.concatenate(blocks, axis=1) + bt2_ref[0, 0])
    xr_ref[0] = xr

    @pl.when(b == 0)
    def _init():
        rec_ref[...] = jnp.zeros((1, 1), jnp.float32)
        com_ref[...] = jnp.zeros((1, 1), jnp.float32)
        cs = cs_ref[...]
        n = cs / (jnp.sum(cs) + 1e-6)
        ppx_ref[...] = jnp.exp(-jnp.sum(n * jnp.log(n + 1e-6),
                                        keepdims=True))

    dr = xr - xf_ref[0]
    rec_ref[...] += jnp.sum(dr * dr, keepdims=True)
    dc = z_ref[0] - zq
    com_ref[...] += jnp.sum(dc * dc, keepdims=True)


def _decoder_call(zq, z, xf, wd1c, bd1, w01, w23, bt1, wt2a, wt2b, bt2, cs):
    full = lambda a: pl.BlockSpec(a.shape, lambda b: (0,) * a.ndim)
    return pl.pallas_call(
        _decoder_body,
        grid=(B,),
        in_specs=[
            pl.BlockSpec((1, TQ, D), lambda b: (b, 0, 0)),
            pl.BlockSpec((1, TQ, D), lambda b: (b, 0, 0)),
            pl.BlockSpec((1, 2048, 16), lambda b: (b, 0, 0)),
            full(wd1c), full(bd1), full(w01), full(w23), full(bt1),
            full(wt2a), full(wt2b), full(bt2), full(cs),
        ],
        out_specs=[
            pl.BlockSpec((1, 2048, 16), lambda b: (b, 0, 0)),
            pl.BlockSpec((1, 1), lambda b: (0, 0)),
            pl.BlockSpec((1, 1), lambda b: (0, 0)),
            pl.BlockSpec((1, 1), lambda b: (0, 0)),
        ],
        out_shape=[
            jax.ShapeDtypeStruct((B, 2048, 16), jnp.float32),
            jax.ShapeDtypeStruct((1, 1), jnp.float32),
            jax.ShapeDtypeStruct((1, 1), jnp.float32),
            jax.ShapeDtypeStruct((1, 1), jnp.float32),
        ],
    )(zq, z, xf, wd1c, bd1, w01, w23, bt1, wt2a, wt2b, bt2, cs)


def kernel(x, enc_w1, enc_b1, enc_w2, enc_b2, enc_w3, enc_b3,
           dec_w1, dec_b1, dec_wt1, dec_bt1, dec_wt2, dec_bt2,
           embedding, cluster_size):
    f32 = jnp.float32

    bf = jnp.bfloat16
    xf = x.reshape(B, 2048, 16)
    xfp = jnp.pad(xf, ((0, 0), (1, 1), (0, 0)))
    w1t = enc_w1[:, 0, :].T.astype(bf)
    w2c = enc_w2.transpose(2, 1, 0).reshape(8 * 128, 128).astype(bf)
    w3c = enc_w3.transpose(2, 1, 0).reshape(3 * 128, 128).astype(bf)
    embT = embedding.T
    b1 = enc_b1.reshape(1, 128)
    b2 = enc_b2.reshape(1, 128)
    b3 = enc_b3.reshape(1, 128)

    wd1c = dec_w1.transpose(2, 1, 0).reshape(3 * 128, 128).astype(bf)
    bd1 = dec_b1.reshape(1, 128)
    w01 = jnp.concatenate(
        [jnp.concatenate([dec_wt1[:, :, 2], dec_wt1[:, :, 3]], axis=1),
         jnp.concatenate([dec_wt1[:, :, 6], dec_wt1[:, :, 7]], axis=1)], axis=0)
    w01 = w01.astype(bf)
    w23 = jnp.concatenate(
        [jnp.concatenate([dec_wt1[:, :, 4], dec_wt1[:, :, 5]], axis=1),
         jnp.concatenate([dec_wt1[:, :, 0], dec_wt1[:, :, 1]], axis=1)], axis=0)
    w23 = w23.astype(bf)
    bt1 = jnp.concatenate([dec_bt1, dec_bt1]).reshape(1, 256)
    wt2a = jnp.stack([dec_wt2[:, 0, 2], dec_wt2[:, 0, 3],
                      dec_wt2[:, 0, 4], dec_wt2[:, 0, 5]], axis=1).astype(bf)
    wt2b = jnp.stack([dec_wt2[:, 0, 6], dec_wt2[:, 0, 7],
                      dec_wt2[:, 0, 0], dec_wt2[:, 0, 1]], axis=1).astype(bf)
    bt2 = dec_bt2.reshape(1, 1)
    cs = cluster_size.reshape(1, K)

    z, idx3 = _encoder_call(xfp, w1t, b1, w2c, b2, w3c, b3, embT)
    idx_flat = idx3.reshape(B * TQ)

    zq_flat = _gather_call(embedding, idx_flat)
    zq = zq_flat.reshape(B, TQ, D)

    xr_f, rec_s, com_s, ppx_s = _decoder_call(
        zq, z, xf, wd1c, bd1, w01, w23, bt1, wt2a, wt2b, bt2, cs)

    xr = xr_f.reshape(B, 1, T)
    rec = (rec_s / f32(B * T)).reshape(())
    com = (com_s / f32(B * D * TQ)).reshape(())
    ppx = ppx_s.reshape(())
    idx = idx3.reshape(B, TQ)
    return (rec, com, ppx, xr, idx)
